# baseline (speedup 1.0000x reference)
import functools
import math

import jax
import jax.numpy as jnp
from jax import lax
from jax.experimental import pallas as pl
from jax.experimental.pallas import tpu as pltpu
from jax.experimental.pallas import tpu_sc as plsc

_SCALE = math.sqrt(32.0)

_NC = 2
_NS = 16
_NW = _NC * _NS

_V = 1000000
_D = 32
_VPAD = 1000064
_NBLK = _V // 128
_TAILW = _V - _NBLK * 128
_TAILWID = _NBLK % _NW


def _do_transpose(tbuf, obuf, width):
    for j in range(_D):
        for g in range(width // 16):
            v = tbuf[j, pl.ds(g * 16, 16)]
            rows = lax.broadcasted_iota(jnp.int32, (16,), 0) + g * 16
            cols = jnp.full((16,), j, jnp.int32)
            plsc.store_scatter(obuf, [rows, cols], v)


def _transpose_kernel(table_hbm, scr_hbm, tbufs, obufs, tailbuf, rsems, wsems):
    wid = lax.axis_index("s") * _NC + lax.axis_index("c")
    niter = _NBLK // _NW + 1

    def start_read(k, slot):
        b = k * _NW + wid

        @pl.when(b < _NBLK)
        def _():
            pltpu.async_copy(table_hbm.at[:, pl.ds(b * 128, 128)],
                             tbufs.at[slot], rsems.at[slot])

    start_read(0, 0)
    start_read(1, 1)

    def pair_body(t, carry):
        for s in range(2):
            k = t * 2 + s
            b = k * _NW + wid

            @pl.when(b < _NBLK)
            def _():
                pltpu.make_async_copy(table_hbm.at[:, pl.ds(0, 128)],
                                      tbufs.at[s], rsems.at[s]).wait()

                @pl.when(k >= 2)
                def _():
                    pltpu.make_async_copy(obufs.at[s, :, pl.ds(0, _D)],
                                          scr_hbm.at[pl.ds(0, 128)],
                                          wsems.at[s]).wait()

                _do_transpose(tbufs.at[s], obufs.at[s], 128)
                pltpu.async_copy(obufs.at[s, :, pl.ds(0, _D)],
                                 scr_hbm.at[pl.ds(b * 128, 128)],
                                 wsems.at[s])
            start_read(k + 2, s)
        return carry

    lax.fori_loop(0, (niter + 1) // 2, pair_body, 0)

    for k in range(max(niter - 3, 0), niter):
        s = k % 2
        b = k * _NW + wid
        b2 = (k + 2) * _NW + wid

        @pl.when((b < _NBLK) & (b2 >= _NBLK))
        def _():
            pltpu.make_async_copy(obufs.at[s, :, pl.ds(0, _D)],
                                  scr_hbm.at[pl.ds(0, 128)],
                                  wsems.at[s]).wait()

    @pl.when(wid == _TAILWID)
    def _():
        pltpu.sync_copy(table_hbm.at[:, pl.ds(_NBLK * 128, _TAILW)], tailbuf)
        _do_transpose(tailbuf, obufs.at[0], _TAILW)
        pltpu.sync_copy(obufs.at[0, pl.ds(0, _TAILW), pl.ds(0, _D)],
                        scr_hbm.at[pl.ds(_NBLK * 128, _TAILW)])


def _gather_kernel(x_hbm, scr_hbm, out_hbm, idx_v, rows_g, rows_s,
                   gsems, wsems):
    wid = lax.axis_index("s") * _NC + lax.axis_index("c")
    base = wid * 512
    pltpu.sync_copy(x_hbm.at[wid], idx_v)

    def start_gather(k, slot):
        pltpu.async_copy(scr_hbm.at[idx_v.at[k]], rows_g.at[slot],
                         gsems.at[slot])

    start_gather(0, 0)
    start_gather(1, 1)

    def pair_body(t, carry):
        for s in range(2):
            k = t * 2 + s
            pltpu.make_async_copy(scr_hbm.at[idx_v.at[0]], rows_g.at[s],
                                  gsems.at[s]).wait()

            @pl.when(k >= 2)
            def _():
                pltpu.make_async_copy(rows_s.at[s], out_hbm.at[base],
                                      wsems.at[s]).wait()

            for i in range(50):
                rows_s[s, i, pl.ds(0, 16)] = rows_g[s, i, pl.ds(0, 16)] * _SCALE
                rows_s[s, i, pl.ds(16, 16)] = (
                    rows_g[s, i, pl.ds(16, 16)] * _SCALE)
            pltpu.async_copy(rows_s.at[s], out_hbm.at[base + k], wsems.at[s])

            @pl.when(k + 2 < 512)
            def _():
                start_gather(k + 2, s)
        return carry

    lax.fori_loop(0, 256, pair_body, 0)
    for s in range(2):
        pltpu.make_async_copy(rows_s.at[s], out_hbm.at[base],
                              wsems.at[s]).wait()


def kernel(x, table):
    b0, s = x.shape
    v, d = table.shape
    xf = x.reshape(_NW, 512, 50).astype(jnp.int32)
    table_t = table.T

    mesh = plsc.VectorSubcoreMesh(
        core_axis_name="c", subcore_axis_name="s", num_cores=_NC,
        num_subcores=_NS)
    params = pltpu.CompilerParams(use_tc_tiling_on_sc=False,
                                  needs_layout_passes=False)

    scr = pl.kernel(
        _transpose_kernel,
        out_type=jax.ShapeDtypeStruct((_VPAD, _D), jnp.float32),
        mesh=mesh,
        compiler_params=params,
        scratch_types=[
            pltpu.VMEM((2, _D, 128), jnp.float32),
            pltpu.VMEM((2, 128, 33), jnp.float32),
            pltpu.VMEM((_D, _TAILW), jnp.float32),
            pltpu.SemaphoreType.DMA((2,)),
            pltpu.SemaphoreType.DMA((2,)),
        ],
    )(table_t)

    out = pl.kernel(
        _gather_kernel,
        out_type=jax.ShapeDtypeStruct((b0, s, d), jnp.float32),
        mesh=mesh,
        compiler_params=params,
        scratch_types=[
            pltpu.VMEM((512, 50), jnp.int32),
            pltpu.VMEM((2, 50, _D), jnp.float32),
            pltpu.VMEM((2, 50, _D), jnp.float32),
            pltpu.SemaphoreType.DMA((2,)),
            pltpu.SemaphoreType.DMA((2,)),
        ],
    )(xf, scr)
    return out


# speedup vs baseline: 3.6221x; 3.6221x over previous
import math

import jax
import jax.numpy as jnp
from jax import lax
from jax.experimental import pallas as pl
from jax.experimental.pallas import tpu as pltpu
from jax.experimental.pallas import tpu_sc as plsc

_SCALE = math.sqrt(32.0)

_NC = 2
_NS = 16
_NW = _NC * _NS

_V = 1000000
_D = 32
_VPAD = 1000064
_NBLK = _V // 128
_TAILW = _V - _NBLK * 128
_TAILWID = _NBLK % _NW
_PITCH = 132


def _scatter_block(tbuf, obuf, width):
    lam = lax.broadcasted_iota(jnp.int32, (16,), 0)
    for g in range(width // 16):
        rows = (lam + g * 16) // 4
        colbase = ((lam + g * 16) % 4) * 32
        for d in range(_D):
            v = tbuf[d, pl.ds(g * 16, 16)]
            plsc.store_scatter(obuf, [rows, colbase + d], v)


def _transpose_kernel(table_hbm, scr_hbm, tbufs, obufs, tailbuf, rsems,
                      wsems):
    wid = lax.axis_index("s") * _NC + lax.axis_index("c")
    niter = _NBLK // _NW + 1

    def start_read(k, slot):
        b = k * _NW + wid

        @pl.when(b < _NBLK)
        def _():
            pltpu.async_copy(table_hbm.at[:, pl.ds(b * 128, 128)],
                             tbufs.at[slot], rsems.at[slot])

    start_read(0, 0)
    start_read(1, 1)

    def pair_body(t, carry):
        for s in range(2):
            k = t * 2 + s
            b = k * _NW + wid

            @pl.when(b < _NBLK)
            def _():
                pltpu.make_async_copy(table_hbm.at[:, pl.ds(0, 128)],
                                      tbufs.at[s], rsems.at[s]).wait()

                @pl.when(k >= 2)
                def _():
                    pltpu.make_async_copy(
                        obufs.at[s, :, pl.ds(0, 128)],
                        scr_hbm.at[pl.ds(0, 32)], wsems.at[s]).wait()

                _scatter_block(tbufs.at[s], obufs.at[s], 128)
                pltpu.async_copy(obufs.at[s, :, pl.ds(0, 128)],
                                 scr_hbm.at[pl.ds(b * 32, 32)], wsems.at[s])
            start_read(k + 2, s)
        return carry

    lax.fori_loop(0, (niter + 1) // 2, pair_body, 0)

    for k in range(max(niter - 3, 0), niter):
        s = k % 2
        b = k * _NW + wid
        b2 = (k + 2) * _NW + wid

        @pl.when((b < _NBLK) & (b2 >= _NBLK))
        def _():
            pltpu.make_async_copy(obufs.at[s, :, pl.ds(0, 128)],
                                  scr_hbm.at[pl.ds(0, 32)],
                                  wsems.at[s]).wait()

    @pl.when(wid == _TAILWID)
    def _():
        pltpu.sync_copy(table_hbm.at[:, pl.ds(_NBLK * 128, _TAILW)], tailbuf)
        _scatter_block(tailbuf, obufs.at[0], _TAILW)
        pltpu.sync_copy(obufs.at[0, pl.ds(0, _TAILW // 4), pl.ds(0, 128)],
                        scr_hbm.at[pl.ds(_NBLK * 32, _TAILW // 4)])


def _gather5_kernel(xt_hbm, scr_hbm, out_hbm, idx_v, rows_g, obufs,
                    gsems, wsems):
    wid = lax.axis_index("s") * _NC + lax.axis_index("c")
    pltpu.sync_copy(xt_hbm.at[:, pl.ds(wid * 512, 512)], idx_v)

    def start_gather(kb, slot):
        s = kb // 4
        q = kb % 4
        pltpu.async_copy(scr_hbm.at[idx_v.at[s, pl.ds(q * 128, 128)]],
                         rows_g.at[slot], gsems.at[slot])

    start_gather(0, 0)
    start_gather(1, 1)

    def pair_body(t, carry):
        for sl in range(2):
            kb = t * 2 + sl
            s = kb // 4
            q = kb % 4
            pltpu.make_async_copy(scr_hbm.at[idx_v.at[0, pl.ds(0, 128)]],
                                  rows_g.at[sl], gsems.at[sl]).wait()

            @pl.when(kb >= 2)
            def _():
                for db in range(4):
                    pltpu.make_async_copy(
                        obufs.at[sl, pl.ds(db * 8, 8), pl.ds(0, 128)],
                        out_hbm.at[0, db, 0], wsems.at[sl]).wait()

            lam = lax.broadcasted_iota(jnp.int32, (16,), 0)
            for n in range(128):
                for h in range(2):
                    v = rows_g[sl, n, pl.ds(h * 16, 16)] * _SCALE
                    plsc.store_scatter(obufs.at[sl],
                                       [lam + h * 16,
                                        jnp.full((16,), n, jnp.int32)], v)
            for db in range(4):
                pltpu.async_copy(obufs.at[sl, pl.ds(db * 8, 8),
                                          pl.ds(0, 128)],
                                 out_hbm.at[s, db, wid * 4 + q],
                                 wsems.at[sl])

            @pl.when(kb + 2 < 200)
            def _():
                start_gather_dyn(kb + 2, sl)
        return carry

    def start_gather_dyn(kb, slot):
        s = kb // 4
        q = kb % 4
        pltpu.async_copy(scr_hbm.at[idx_v.at[s, pl.ds(q * 128, 128)]],
                         rows_g.at[slot], gsems.at[slot])

    lax.fori_loop(0, 100, pair_body, 0)
    for sl in range(2):
        for db in range(4):
            pltpu.make_async_copy(obufs.at[sl, pl.ds(db * 8, 8),
                                           pl.ds(0, 128)],
                                  out_hbm.at[0, db, 0], wsems.at[sl]).wait()


def kernel(x, table):
    b0, s = x.shape
    v, d = table.shape
    xt = x.T.astype(jnp.int32)
    table_t = table.T

    mesh = plsc.VectorSubcoreMesh(
        core_axis_name="c", subcore_axis_name="s", num_cores=_NC,
        num_subcores=_NS)

    scr128 = pl.kernel(
        _transpose_kernel,
        out_type=jax.ShapeDtypeStruct((_VPAD // 4, 128), jnp.float32),
        mesh=mesh,
        compiler_params=pltpu.CompilerParams(use_tc_tiling_on_sc=True,
                                             needs_layout_passes=False),
        scratch_types=[
            pltpu.VMEM((2, _D, 128), jnp.float32),
            pltpu.VMEM((2, _D, _PITCH), jnp.float32),
            pltpu.VMEM((_D, _TAILW), jnp.float32),
            pltpu.SemaphoreType.DMA((2,)),
            pltpu.SemaphoreType.DMA((2,)),
        ],
    )(table_t)

    out5 = pl.kernel(
        _gather5_kernel,
        out_type=jax.ShapeDtypeStruct((50, 4, 128, 8, 128), jnp.float32),
        mesh=mesh,
        compiler_params=pltpu.CompilerParams(use_tc_tiling_on_sc=False,
                                             needs_layout_passes=False),
        scratch_types=[
            pltpu.VMEM((50, 512), jnp.int32),
            pltpu.VMEM((2, 128, _D), jnp.float32),
            pltpu.VMEM((2, _D, _PITCH), jnp.float32),
            pltpu.SemaphoreType.DMA((2,)),
            pltpu.SemaphoreType.DMA((2,)),
        ],
    )(xt, scr128.reshape(_VPAD, _D))
    return out5.transpose(2, 4, 0, 1, 3).reshape(b0, s, d)


# speedup vs baseline: 3.6239x; 1.0005x over previous
import math

import jax
import jax.numpy as jnp
from jax import lax
from jax.experimental import pallas as pl
from jax.experimental.pallas import tpu as pltpu
from jax.experimental.pallas import tpu_sc as plsc

_SCALE = math.sqrt(32.0)

_NC = 2
_NS = 16
_NW = _NC * _NS

_V = 1000000
_D = 32
_VPAD = 1000064
_NBLK = _V // 128
_TAILW = _V - _NBLK * 128
_TAILWID = _NBLK % _NW
_PITCH = 132


def _scatter_block(tbuf, obuf, width):
    lam = lax.broadcasted_iota(jnp.int32, (16,), 0)
    for g in range(width // 16):
        rows = (lam + g * 16) // 4
        colbase = ((lam + g * 16) % 4) * 32
        for d in range(_D):
            v = tbuf[d, pl.ds(g * 16, 16)]
            plsc.store_scatter(obuf, [rows, colbase + d], v)


def _transpose_kernel(table_hbm, scr_hbm, tbufs, obufs, tailbuf, rsems,
                      wsems):
    wid = lax.axis_index("s") * _NC + lax.axis_index("c")
    niter = _NBLK // _NW + 1

    def start_read(k, slot):
        b = k * _NW + wid

        @pl.when(b < _NBLK)
        def _():
            pltpu.async_copy(table_hbm.at[:, pl.ds(b * 128, 128)],
                             tbufs.at[slot], rsems.at[slot])

    start_read(0, 0)
    start_read(1, 1)

    def pair_body(t, carry):
        for s in range(2):
            k = t * 2 + s
            b = k * _NW + wid

            @pl.when(b < _NBLK)
            def _():
                pltpu.make_async_copy(table_hbm.at[:, pl.ds(0, 128)],
                                      tbufs.at[s], rsems.at[s]).wait()

                @pl.when(k >= 2)
                def _():
                    pltpu.make_async_copy(
                        obufs.at[s, :, pl.ds(0, 128)],
                        scr_hbm.at[pl.ds(0, 32)], wsems.at[s]).wait()

                _scatter_block(tbufs.at[s], obufs.at[s], 128)
                pltpu.async_copy(obufs.at[s, :, pl.ds(0, 128)],
                                 scr_hbm.at[pl.ds(b * 32, 32)], wsems.at[s])
            start_read(k + 2, s)
        return carry

    lax.fori_loop(0, (niter + 1) // 2, pair_body, 0)

    for k in range(max(niter - 3, 0), niter):
        s = k % 2
        b = k * _NW + wid
        b2 = (k + 2) * _NW + wid

        @pl.when((b < _NBLK) & (b2 >= _NBLK))
        def _():
            pltpu.make_async_copy(obufs.at[s, :, pl.ds(0, 128)],
                                  scr_hbm.at[pl.ds(0, 32)],
                                  wsems.at[s]).wait()

    @pl.when(wid == _TAILWID)
    def _():
        pltpu.sync_copy(table_hbm.at[:, pl.ds(_NBLK * 128, _TAILW)], tailbuf)
        _scatter_block(tailbuf, obufs.at[0], _TAILW)
        pltpu.sync_copy(obufs.at[0, pl.ds(0, _TAILW // 4), pl.ds(0, 128)],
                        scr_hbm.at[pl.ds(_NBLK * 32, _TAILW // 4)])


def _gather5_kernel(xt_hbm, scr_hbm, out_hbm, idx_v, rows_g, obufs,
                    gsems, wsems):
    wid = lax.axis_index("s") * _NC + lax.axis_index("c")
    pltpu.sync_copy(xt_hbm.at[:, pl.ds(wid * 512, 512)], idx_v)

    def start_gather(kb, slot):
        s = kb // 4
        q = kb % 4
        pltpu.async_copy(scr_hbm.at[idx_v.at[s, pl.ds(q * 128, 128)]],
                         rows_g.at[slot], gsems.at[slot])

    start_gather(0, 0)
    start_gather(1, 1)

    def pair_body(t, carry):
        for sl in range(2):
            kb = t * 2 + sl
            s = kb // 4
            q = kb % 4
            pltpu.make_async_copy(scr_hbm.at[idx_v.at[0, pl.ds(0, 128)]],
                                  rows_g.at[sl], gsems.at[sl]).wait()

            @pl.when(kb >= 2)
            def _():
                for db in range(4):
                    pltpu.make_async_copy(
                        obufs.at[sl, pl.ds(db * 8, 8), pl.ds(0, 128)],
                        out_hbm.at[0, db, 0], wsems.at[sl]).wait()

            lam = lax.broadcasted_iota(jnp.int32, (16,), 0)
            for n in range(128):
                for h in range(2):
                    v = rows_g[sl, n, pl.ds(h * 16, 16)] * _SCALE
                    plsc.store_scatter(obufs.at[sl],
                                       [lam + h * 16,
                                        jnp.full((16,), n, jnp.int32)], v)
            for db in range(4):
                pltpu.async_copy(obufs.at[sl, pl.ds(db * 8, 8),
                                          pl.ds(0, 128)],
                                 out_hbm.at[s, db, wid * 4 + q],
                                 wsems.at[sl])

            @pl.when(kb + 2 < 200)
            def _():
                start_gather_dyn(kb + 2, sl)
        return carry

    def start_gather_dyn(kb, slot):
        s = kb // 4
        q = kb % 4
        pltpu.async_copy(scr_hbm.at[idx_v.at[s, pl.ds(q * 128, 128)]],
                         rows_g.at[slot], gsems.at[slot])

    lax.fori_loop(0, 100, pair_body, 0)
    for sl in range(2):
        for db in range(4):
            pltpu.make_async_copy(obufs.at[sl, pl.ds(db * 8, 8),
                                           pl.ds(0, 128)],
                                  out_hbm.at[0, db, 0], wsems.at[sl]).wait()


def kernel(x, table):
    b0, s = x.shape
    v, d = table.shape
    xt = x.T.astype(jnp.int32)
    table_t = table.T

    mesh = plsc.VectorSubcoreMesh(
        core_axis_name="c", subcore_axis_name="s", num_cores=_NC,
        num_subcores=_NS)

    scr128 = pl.kernel(
        _transpose_kernel,
        out_type=jax.ShapeDtypeStruct((_VPAD // 4, 128), jnp.float32),
        mesh=mesh,
        compiler_params=pltpu.CompilerParams(use_tc_tiling_on_sc=True,
                                             needs_layout_passes=False),
        scratch_types=[
            pltpu.VMEM((2, _D, 128), jnp.float32),
            pltpu.VMEM((2, _D, _PITCH), jnp.float32),
            pltpu.VMEM((_D, _TAILW), jnp.float32),
            pltpu.SemaphoreType.DMA((2,)),
            pltpu.SemaphoreType.DMA((2,)),
        ],
    )(table_t)

    out5 = pl.kernel(
        _gather5_kernel,
        out_type=jax.ShapeDtypeStruct((50, 4, 128, 8, 128), jnp.float32),
        mesh=mesh,
        compiler_params=pltpu.CompilerParams(use_tc_tiling_on_sc=False,
                                             needs_layout_passes=False),
        scratch_types=[
            pltpu.VMEM((50, 512), jnp.int32),
            pltpu.VMEM((2, 128, _D), jnp.float32),
            pltpu.VMEM((2, _D, 131), jnp.float32),
            pltpu.SemaphoreType.DMA((2,)),
            pltpu.SemaphoreType.DMA((2,)),
        ],
    )(xt, scr128.reshape(_VPAD, _D))
    return out5.transpose(2, 4, 0, 1, 3).reshape(b0, s, d)
